# baseline (speedup 1.0000x reference)
import functools

import jax
import jax.numpy as jnp
from jax import lax
from jax.experimental import pallas as pl
from jax.experimental.pallas import tpu as pltpu
from jax.experimental.pallas import tpu_sc as plsc

N = 10000
E = 320000
NW = 32
NT = 16
CH = 128
CPW = 80
E_PAD = NW * CPW * CH
N_ACC = 10240
ROWS_PER_TILE = N_ACC // NT
NCOPY = ROWS_PER_TILE // CH
DUMP = 10016
DEG_W = 8


def _mesh():
    return plsc.VectorSubcoreMesh(core_axis_name="c", subcore_axis_name="s")


@functools.cache
def _prop_kernel(d):
    @functools.partial(
        pl.kernel,
        out_type=jax.ShapeDtypeStruct((2, N_ACC, d), jnp.float32),
        mesh=_mesh(),
        scratch_types=[
            pltpu.VMEM((CPW, CH), jnp.int32),
            pltpu.VMEM((CPW, CH), jnp.int32),
            pltpu.VMEM((CH, d), jnp.float32),
            pltpu.VMEM((CH, d), jnp.float32),
            pltpu.VMEM_SHARED((N_ACC, d), jnp.float32),
            pltpu.SemaphoreType.DMA,
            pltpu.SemaphoreType.DMA,
        ],
        compiler_params=pltpu.CompilerParams(use_tc_tiling_on_sc=False),
    )
    def prop(hs, srcp, dstp, zrow, out, src_v, dst_v, rows0, rows1, acc, sem0, sem1):
        c = lax.axis_index("c")
        t = lax.axis_index("s")
        wid = c * NT + t
        pltpu.sync_copy(zrow, rows0)
        for j in range(NCOPY):
            pltpu.sync_copy(rows0, acc.at[pl.ds((t * NCOPY + j) * CH, CH)])
        plsc.subcore_barrier()
        pltpu.sync_copy(srcp.at[wid], src_v)
        pltpu.sync_copy(dstp.at[wid], dst_v)

        pltpu.async_copy(hs.at[src_v.at[0]], rows0, sem0)

        def body(j2, carry):
            j = 2 * j2
            pltpu.make_async_copy(hs.at[src_v.at[j]], rows0, sem0).wait()
            pltpu.async_copy(hs.at[src_v.at[j + 1]], rows1, sem1)
            pltpu.sync_copy(rows0, acc.at[dst_v.at[j]], add=True)
            pltpu.make_async_copy(hs.at[src_v.at[j + 1]], rows1, sem1).wait()

            @pl.when(j + 2 < CPW)
            def _():
                pltpu.async_copy(hs.at[src_v.at[j + 2]], rows0, sem0)

            pltpu.sync_copy(rows1, acc.at[dst_v.at[j + 1]], add=True)
            return carry

        lax.fori_loop(0, CPW // 2, body, 0)
        plsc.subcore_barrier()
        for j in range(NCOPY):
            off = (t * NCOPY + j) * CH
            pltpu.sync_copy(acc.at[pl.ds(off, CH)], rows0)
            pltpu.sync_copy(rows0, out.at[c, pl.ds(off, CH)])

    return prop


@functools.cache
def _deg_kernel():
    @functools.partial(
        pl.kernel,
        out_type=jax.ShapeDtypeStruct((2, N_ACC, DEG_W), jnp.float32),
        mesh=_mesh(),
        scratch_types=[
            pltpu.VMEM((CPW, CH), jnp.int32),
            pltpu.VMEM((CH, DEG_W), jnp.float32),
            pltpu.VMEM((CH, DEG_W), jnp.float32),
            pltpu.VMEM_SHARED((N_ACC, DEG_W), jnp.float32),
        ],
        compiler_params=pltpu.CompilerParams(use_tc_tiling_on_sc=False),
    )
    def degk(dstp, ones_hbm, zrow, out, dst_v, ones_v, buf_v, acc):
        c = lax.axis_index("c")
        t = lax.axis_index("s")
        wid = c * NT + t
        pltpu.sync_copy(zrow, buf_v)
        for j in range(NCOPY):
            pltpu.sync_copy(buf_v, acc.at[pl.ds((t * NCOPY + j) * CH, CH)])
        plsc.subcore_barrier()
        pltpu.sync_copy(dstp.at[wid], dst_v)
        pltpu.sync_copy(ones_hbm, ones_v)

        def body(j, carry):
            pltpu.sync_copy(ones_v, acc.at[dst_v.at[j]], add=True)
            return carry

        lax.fori_loop(0, CPW, body, 0)
        plsc.subcore_barrier()
        for j in range(NCOPY):
            off = (t * NCOPY + j) * CH
            pltpu.sync_copy(acc.at[pl.ds(off, CH)], buf_v)
            pltpu.sync_copy(buf_v, out.at[c, pl.ds(off, CH)])

    return degk


def _tc1(x, W1, dp):
    def body(x_ref, w_ref, dp_ref, ys_ref, s8_ref):
        deg = (dp_ref[0] + dp_ref[1])[:N]
        s8 = jnp.where(deg > 0, lax.rsqrt(jnp.maximum(deg, 1e-12)), 0.0)
        s8_ref[...] = s8
        y = jnp.dot(x_ref[...], w_ref[...], preferred_element_type=jnp.float32)
        ys_ref[...] = y * s8[:, :1]

    return pl.pallas_call(
        body,
        out_shape=(
            jax.ShapeDtypeStruct((N, 64), jnp.float32),
            jax.ShapeDtypeStruct((N, DEG_W), jnp.float32),
        ),
    )(x, W1, dp)


def _tc_combine(p, s8, d):
    def body(p_ref, s8_ref, z_ref, zs_ref):
        s = s8_ref[...][:, :1]
        z = s * (p_ref[0] + p_ref[1])[:N]
        z_ref[...] = z
        zs_ref[...] = s * z

    return pl.pallas_call(
        body,
        out_shape=(
            jax.ShapeDtypeStruct((N, d), jnp.float32),
            jax.ShapeDtypeStruct((N, d), jnp.float32),
        ),
    )(p, s8)


def _tc_mid(q, z1, s8, b1, W2, aa):
    def body(q_ref, z1_ref, s8_ref, b1_ref, w2_ref, aa_ref, us_ref):
        s = s8_ref[...][:, :1]
        z2 = s * (q_ref[0] + q_ref[1])[:N]
        h = jnp.maximum(0.5 * z1_ref[...] + (0.25 * aa_ref[0]) * z2 + b1_ref[...], 0.0)
        u = jnp.dot(h, w2_ref[...], preferred_element_type=jnp.float32)
        us_ref[...] = s * u

    return pl.pallas_call(
        body,
        in_specs=[
            pl.BlockSpec(memory_space=pltpu.VMEM),
            pl.BlockSpec(memory_space=pltpu.VMEM),
            pl.BlockSpec(memory_space=pltpu.VMEM),
            pl.BlockSpec(memory_space=pltpu.VMEM),
            pl.BlockSpec(memory_space=pltpu.VMEM),
            pl.BlockSpec(memory_space=pltpu.SMEM),
        ],
        out_shape=jax.ShapeDtypeStruct((N, 16), jnp.float32),
    )(q, z1, s8, b1, W2, aa)


def _tc_final(t, v1, s8, b2, aa):
    def body(t_ref, v1_ref, s8_ref, b2_ref, aa_ref, o_ref):
        s = s8_ref[...][:, :1]
        v2 = s * (t_ref[0] + t_ref[1])[:N]
        o = 0.5 * v1_ref[...] + (0.25 * aa_ref[0]) * v2 + b2_ref[...]
        m = jnp.max(o, axis=1, keepdims=True)
        lse = jnp.log(jnp.sum(jnp.exp(o - m), axis=1, keepdims=True)) + m
        o_ref[...] = o - lse

    return pl.pallas_call(
        body,
        in_specs=[
            pl.BlockSpec(memory_space=pltpu.VMEM),
            pl.BlockSpec(memory_space=pltpu.VMEM),
            pl.BlockSpec(memory_space=pltpu.VMEM),
            pl.BlockSpec(memory_space=pltpu.VMEM),
            pl.BlockSpec(memory_space=pltpu.SMEM),
        ],
        out_shape=jax.ShapeDtypeStruct((N, 16), jnp.float32),
    )(t, v1, s8, b2, aa)


def kernel(x, edge_index, W1, b1, W2, b2, att_w, att_b):
    src = edge_index[0].astype(jnp.int32)
    dst = edge_index[1].astype(jnp.int32)
    pad = E_PAD - E
    srcp = jnp.concatenate([src, jnp.zeros((pad,), jnp.int32)]).reshape(NW, CPW, CH)
    dstp = jnp.concatenate([dst, jnp.full((pad,), DUMP, jnp.int32)]).reshape(NW, CPW, CH)

    a = jax.nn.sigmoid(jax.nn.relu(att_w[0, 0] + att_w[1, 0]) + att_b[0])
    aa = (a * a).reshape(1).astype(jnp.float32)

    zeros64 = jnp.zeros((CH, 64), jnp.float32)
    zeros16 = jnp.zeros((CH, 16), jnp.float32)
    zeros8 = jnp.zeros((CH, DEG_W), jnp.float32)
    ones8 = jnp.ones((CH, DEG_W), jnp.float32)

    dp = _deg_kernel()(dstp, ones8, zeros8)
    ys, s8 = _tc1(x, W1, dp)

    p = _prop_kernel(64)(ys, srcp, dstp, zeros64)
    z1, ys2 = _tc_combine(p, s8, 64)
    q = _prop_kernel(64)(ys2, srcp, dstp, zeros64)
    us = _tc_mid(q, z1, s8, b1.reshape(1, 64), W2, aa)

    r = _prop_kernel(16)(us, srcp, dstp, zeros16)
    v1, us2 = _tc_combine(r, s8, 16)
    t = _prop_kernel(16)(us2, srcp, dstp, zeros16)
    return _tc_final(t, v1, s8, b2.reshape(1, 16), aa)


# speedup vs baseline: 1.0273x; 1.0273x over previous
import functools

import jax
import jax.numpy as jnp
from jax import lax
from jax.experimental import pallas as pl
from jax.experimental.pallas import tpu as pltpu
from jax.experimental.pallas import tpu_sc as plsc

N = 10000
E = 320000
NW = 32
NT = 16
CH = 128
CPW = 80
E_PAD = NW * CPW * CH
N_ACC = 10240
ROWS_PER_TILE = N_ACC // NT
NCOPY = ROWS_PER_TILE // CH
DUMP = 10016
DEG_W = 8
NBUF = 4


def _mesh():
    return plsc.VectorSubcoreMesh(core_axis_name="c", subcore_axis_name="s")


@functools.cache
def _prop_kernel(d):
    @functools.partial(
        pl.kernel,
        out_type=jax.ShapeDtypeStruct((2, N_ACC, d), jnp.float32),
        mesh=_mesh(),
        scratch_types=[
            pltpu.VMEM((CPW, CH), jnp.int32),
            pltpu.VMEM((CPW, CH), jnp.int32),
            pltpu.VMEM((NBUF, CH, d), jnp.float32),
            pltpu.VMEM_SHARED((N_ACC, d), jnp.float32),
        ] + [pltpu.SemaphoreType.DMA] * NBUF,
        compiler_params=pltpu.CompilerParams(use_tc_tiling_on_sc=False),
    )
    def prop(hs, srcp, dstp, zrow, out, src_v, dst_v, rows, acc, *sems):
        c = lax.axis_index("c")
        t = lax.axis_index("s")
        wid = c * NT + t
        pltpu.sync_copy(zrow, rows.at[0])
        for j in range(NCOPY):
            pltpu.sync_copy(rows.at[0], acc.at[pl.ds((t * NCOPY + j) * CH, CH)])
        plsc.subcore_barrier()
        pltpu.sync_copy(srcp.at[wid], src_v)
        pltpu.sync_copy(dstp.at[wid], dst_v)

        def body(g, carry):
            j = NBUF * g
            cps = [
                pltpu.async_copy(hs.at[src_v.at[j + b]], rows.at[b], sems[b])
                for b in range(NBUF)
            ]
            for b in range(NBUF):
                cps[b].wait()
                pltpu.sync_copy(rows.at[b], acc.at[dst_v.at[j + b]], add=True)
            return carry

        lax.fori_loop(0, CPW // NBUF, body, 0)
        plsc.subcore_barrier()
        for j in range(NCOPY):
            off = (t * NCOPY + j) * CH
            pltpu.sync_copy(acc.at[pl.ds(off, CH)], rows.at[0])
            pltpu.sync_copy(rows.at[0], out.at[c, pl.ds(off, CH)])

    return prop


@functools.cache
def _deg_kernel():
    @functools.partial(
        pl.kernel,
        out_type=jax.ShapeDtypeStruct((2, N_ACC, DEG_W), jnp.float32),
        mesh=_mesh(),
        scratch_types=[
            pltpu.VMEM((CPW, CH), jnp.int32),
            pltpu.VMEM((CH, DEG_W), jnp.float32),
            pltpu.VMEM((CH, DEG_W), jnp.float32),
            pltpu.VMEM_SHARED((N_ACC, DEG_W), jnp.float32),
        ],
        compiler_params=pltpu.CompilerParams(use_tc_tiling_on_sc=False),
    )
    def degk(dstp, ones_hbm, zrow, out, dst_v, ones_v, buf_v, acc):
        c = lax.axis_index("c")
        t = lax.axis_index("s")
        wid = c * NT + t
        pltpu.sync_copy(zrow, buf_v)
        for j in range(NCOPY):
            pltpu.sync_copy(buf_v, acc.at[pl.ds((t * NCOPY + j) * CH, CH)])
        plsc.subcore_barrier()
        pltpu.sync_copy(dstp.at[wid], dst_v)
        pltpu.sync_copy(ones_hbm, ones_v)

        def body(j, carry):
            pltpu.sync_copy(ones_v, acc.at[dst_v.at[j]], add=True)
            return carry

        lax.fori_loop(0, CPW, body, 0)
        plsc.subcore_barrier()
        for j in range(NCOPY):
            off = (t * NCOPY + j) * CH
            pltpu.sync_copy(acc.at[pl.ds(off, CH)], buf_v)
            pltpu.sync_copy(buf_v, out.at[c, pl.ds(off, CH)])

    return degk


def _tc1(x, W1, dp):
    def body(x_ref, w_ref, dp_ref, ys_ref, s8_ref):
        deg = (dp_ref[0] + dp_ref[1])[:N]
        s8 = jnp.where(deg > 0, lax.rsqrt(jnp.maximum(deg, 1e-12)), 0.0)
        s8_ref[...] = s8
        y = jnp.dot(x_ref[...], w_ref[...], preferred_element_type=jnp.float32)
        ys_ref[...] = y * s8[:, :1]

    return pl.pallas_call(
        body,
        out_shape=(
            jax.ShapeDtypeStruct((N, 64), jnp.float32),
            jax.ShapeDtypeStruct((N, DEG_W), jnp.float32),
        ),
    )(x, W1, dp)


def _tc_combine(p, s8, d):
    def body(p_ref, s8_ref, z_ref, zs_ref):
        s = s8_ref[...][:, :1]
        z = s * (p_ref[0] + p_ref[1])[:N]
        z_ref[...] = z
        zs_ref[...] = s * z

    return pl.pallas_call(
        body,
        out_shape=(
            jax.ShapeDtypeStruct((N, d), jnp.float32),
            jax.ShapeDtypeStruct((N, d), jnp.float32),
        ),
    )(p, s8)


def _tc_mid(q, z1, s8, b1, W2, aa):
    def body(q_ref, z1_ref, s8_ref, b1_ref, w2_ref, aa_ref, us_ref):
        s = s8_ref[...][:, :1]
        z2 = s * (q_ref[0] + q_ref[1])[:N]
        h = jnp.maximum(0.5 * z1_ref[...] + (0.25 * aa_ref[0]) * z2 + b1_ref[...], 0.0)
        u = jnp.dot(h, w2_ref[...], preferred_element_type=jnp.float32)
        us_ref[...] = s * u

    return pl.pallas_call(
        body,
        in_specs=[
            pl.BlockSpec(memory_space=pltpu.VMEM),
            pl.BlockSpec(memory_space=pltpu.VMEM),
            pl.BlockSpec(memory_space=pltpu.VMEM),
            pl.BlockSpec(memory_space=pltpu.VMEM),
            pl.BlockSpec(memory_space=pltpu.VMEM),
            pl.BlockSpec(memory_space=pltpu.SMEM),
        ],
        out_shape=jax.ShapeDtypeStruct((N, 16), jnp.float32),
    )(q, z1, s8, b1, W2, aa)


def _tc_final(t, v1, s8, b2, aa):
    def body(t_ref, v1_ref, s8_ref, b2_ref, aa_ref, o_ref):
        s = s8_ref[...][:, :1]
        v2 = s * (t_ref[0] + t_ref[1])[:N]
        o = 0.5 * v1_ref[...] + (0.25 * aa_ref[0]) * v2 + b2_ref[...]
        m = jnp.max(o, axis=1, keepdims=True)
        lse = jnp.log(jnp.sum(jnp.exp(o - m), axis=1, keepdims=True)) + m
        o_ref[...] = o - lse

    return pl.pallas_call(
        body,
        in_specs=[
            pl.BlockSpec(memory_space=pltpu.VMEM),
            pl.BlockSpec(memory_space=pltpu.VMEM),
            pl.BlockSpec(memory_space=pltpu.VMEM),
            pl.BlockSpec(memory_space=pltpu.VMEM),
            pl.BlockSpec(memory_space=pltpu.SMEM),
        ],
        out_shape=jax.ShapeDtypeStruct((N, 16), jnp.float32),
    )(t, v1, s8, b2, aa)


def kernel(x, edge_index, W1, b1, W2, b2, att_w, att_b):
    src = edge_index[0].astype(jnp.int32)
    dst = edge_index[1].astype(jnp.int32)
    pad = E_PAD - E
    srcp = jnp.concatenate([src, jnp.zeros((pad,), jnp.int32)]).reshape(NW, CPW, CH)
    dstp = jnp.concatenate([dst, jnp.full((pad,), DUMP, jnp.int32)]).reshape(NW, CPW, CH)

    a = jax.nn.sigmoid(jax.nn.relu(att_w[0, 0] + att_w[1, 0]) + att_b[0])
    aa = (a * a).reshape(1).astype(jnp.float32)

    zeros64 = jnp.zeros((CH, 64), jnp.float32)
    zeros16 = jnp.zeros((CH, 16), jnp.float32)
    zeros8 = jnp.zeros((CH, DEG_W), jnp.float32)
    ones8 = jnp.ones((CH, DEG_W), jnp.float32)

    dp = _deg_kernel()(dstp, ones8, zeros8)
    ys, s8 = _tc1(x, W1, dp)

    p = _prop_kernel(64)(ys, srcp, dstp, zeros64)
    z1, ys2 = _tc_combine(p, s8, 64)
    q = _prop_kernel(64)(ys2, srcp, dstp, zeros64)
    us = _tc_mid(q, z1, s8, b1.reshape(1, 64), W2, aa)

    r = _prop_kernel(16)(us, srcp, dstp, zeros16)
    v1, us2 = _tc_combine(r, s8, 16)
    t = _prop_kernel(16)(us2, srcp, dstp, zeros16)
    return _tc_final(t, v1, s8, b2.reshape(1, 16), aa)


# speedup vs baseline: 1.0868x; 1.0578x over previous
import functools

import jax
import jax.numpy as jnp
from jax import lax
from jax.experimental import pallas as pl
from jax.experimental.pallas import tpu as pltpu
from jax.experimental.pallas import tpu_sc as plsc

N = 10000
E = 320000
NW = 32
NT = 16
CH = 128
CPW = 80
E_PAD = NW * CPW * CH
N_ACC = 10240
ROWS_PER_TILE = N_ACC // NT
NCOPY = ROWS_PER_TILE // CH
DUMP = 10016
DEG_W = 8
EPW = CPW * CH
S = 1024


def _mesh():
    return plsc.VectorSubcoreMesh(core_axis_name="c", subcore_axis_name="s")


@functools.cache
def _prop_kernel(d):
    @functools.partial(
        pl.kernel,
        out_type=jax.ShapeDtypeStruct((2, N_ACC, d), jnp.float32),
        mesh=_mesh(),
        scratch_types=[
            pltpu.VMEM((EPW,), jnp.int32),
            pltpu.VMEM((EPW,), jnp.int32),
            pltpu.VMEM((S, d), jnp.float32),
            pltpu.VMEM_SHARED((N_ACC, d), jnp.float32),
            pltpu.SemaphoreType.DMA,
        ],
        compiler_params=pltpu.CompilerParams(use_tc_tiling_on_sc=False),
    )
    def prop(hs, srcp, dstp, ztile, out, src_v, dst_v, rows, acc, sem):
        c = lax.axis_index("c")
        t = lax.axis_index("s")
        wid = c * NT + t
        pltpu.sync_copy(ztile, acc.at[pl.ds(t * ROWS_PER_TILE, ROWS_PER_TILE)])
        plsc.subcore_barrier()
        pltpu.sync_copy(srcp.at[wid], src_v)
        pltpu.sync_copy(dstp.at[wid], dst_v)

        def body(g, carry):
            j = S * g
            pltpu.async_copy(hs.at[src_v.at[pl.ds(j, S)]], rows, sem).wait()
            pltpu.sync_copy(rows, acc.at[dst_v.at[pl.ds(j, S)]], add=True)
            return carry

        lax.fori_loop(0, EPW // S, body, 0)
        plsc.subcore_barrier()
        off = t * ROWS_PER_TILE
        pltpu.sync_copy(acc.at[pl.ds(off, ROWS_PER_TILE)],
                        out.at[c, pl.ds(off, ROWS_PER_TILE)])

    return prop


@functools.cache
def _deg_kernel():
    @functools.partial(
        pl.kernel,
        out_type=jax.ShapeDtypeStruct((2, N_ACC, DEG_W), jnp.float32),
        mesh=_mesh(),
        scratch_types=[
            pltpu.VMEM((EPW,), jnp.int32),
            pltpu.VMEM((S, DEG_W), jnp.float32),
            pltpu.VMEM_SHARED((N_ACC, DEG_W), jnp.float32),
        ],
        compiler_params=pltpu.CompilerParams(use_tc_tiling_on_sc=False),
    )
    def degk(dstp, ones_hbm, ztile, out, dst_v, ones_v, acc):
        c = lax.axis_index("c")
        t = lax.axis_index("s")
        wid = c * NT + t
        pltpu.sync_copy(ztile, acc.at[pl.ds(t * ROWS_PER_TILE, ROWS_PER_TILE)])
        plsc.subcore_barrier()
        pltpu.sync_copy(dstp.at[wid], dst_v)
        pltpu.sync_copy(ones_hbm, ones_v)

        def body(g, carry):
            pltpu.sync_copy(ones_v, acc.at[dst_v.at[pl.ds(S * g, S)]], add=True)
            return carry

        lax.fori_loop(0, EPW // S, body, 0)
        plsc.subcore_barrier()
        off = t * ROWS_PER_TILE
        pltpu.sync_copy(acc.at[pl.ds(off, ROWS_PER_TILE)],
                        out.at[c, pl.ds(off, ROWS_PER_TILE)])

    return degk


def _tc1(x, W1, dp):
    def body(x_ref, w_ref, dp_ref, ys_ref, s8_ref):
        deg = (dp_ref[0] + dp_ref[1])[:N]
        s8 = jnp.where(deg > 0, lax.rsqrt(jnp.maximum(deg, 1e-12)), 0.0)
        s8_ref[...] = s8
        y = jnp.dot(x_ref[...], w_ref[...], preferred_element_type=jnp.float32)
        ys_ref[...] = y * s8[:, :1]

    return pl.pallas_call(
        body,
        out_shape=(
            jax.ShapeDtypeStruct((N, 64), jnp.float32),
            jax.ShapeDtypeStruct((N, DEG_W), jnp.float32),
        ),
    )(x, W1, dp)


def _tc_combine(p, s8, d):
    def body(p_ref, s8_ref, z_ref, zs_ref):
        s = s8_ref[...][:, :1]
        z = s * (p_ref[0] + p_ref[1])[:N]
        z_ref[...] = z
        zs_ref[...] = s * z

    return pl.pallas_call(
        body,
        out_shape=(
            jax.ShapeDtypeStruct((N, d), jnp.float32),
            jax.ShapeDtypeStruct((N, d), jnp.float32),
        ),
    )(p, s8)


def _tc_mid(q, z1, s8, b1, W2, aa):
    def body(q_ref, z1_ref, s8_ref, b1_ref, w2_ref, aa_ref, us_ref):
        s = s8_ref[...][:, :1]
        z2 = s * (q_ref[0] + q_ref[1])[:N]
        h = jnp.maximum(0.5 * z1_ref[...] + (0.25 * aa_ref[0]) * z2 + b1_ref[...], 0.0)
        u = jnp.dot(h, w2_ref[...], preferred_element_type=jnp.float32)
        us_ref[...] = s * u

    return pl.pallas_call(
        body,
        in_specs=[
            pl.BlockSpec(memory_space=pltpu.VMEM),
            pl.BlockSpec(memory_space=pltpu.VMEM),
            pl.BlockSpec(memory_space=pltpu.VMEM),
            pl.BlockSpec(memory_space=pltpu.VMEM),
            pl.BlockSpec(memory_space=pltpu.VMEM),
            pl.BlockSpec(memory_space=pltpu.SMEM),
        ],
        out_shape=jax.ShapeDtypeStruct((N, 16), jnp.float32),
    )(q, z1, s8, b1, W2, aa)


def _tc_final(t, v1, s8, b2, aa):
    def body(t_ref, v1_ref, s8_ref, b2_ref, aa_ref, o_ref):
        s = s8_ref[...][:, :1]
        v2 = s * (t_ref[0] + t_ref[1])[:N]
        o = 0.5 * v1_ref[...] + (0.25 * aa_ref[0]) * v2 + b2_ref[...]
        m = jnp.max(o, axis=1, keepdims=True)
        lse = jnp.log(jnp.sum(jnp.exp(o - m), axis=1, keepdims=True)) + m
        o_ref[...] = o - lse

    return pl.pallas_call(
        body,
        in_specs=[
            pl.BlockSpec(memory_space=pltpu.VMEM),
            pl.BlockSpec(memory_space=pltpu.VMEM),
            pl.BlockSpec(memory_space=pltpu.VMEM),
            pl.BlockSpec(memory_space=pltpu.VMEM),
            pl.BlockSpec(memory_space=pltpu.SMEM),
        ],
        out_shape=jax.ShapeDtypeStruct((N, 16), jnp.float32),
    )(t, v1, s8, b2, aa)


def kernel(x, edge_index, W1, b1, W2, b2, att_w, att_b):
    src = edge_index[0].astype(jnp.int32)
    dst = edge_index[1].astype(jnp.int32)
    pad = E_PAD - E
    srcp = jnp.concatenate([src, jnp.zeros((pad,), jnp.int32)]).reshape(NW, EPW)
    dstp = jnp.concatenate([dst, jnp.full((pad,), DUMP, jnp.int32)]).reshape(NW, EPW)

    a = jax.nn.sigmoid(jax.nn.relu(att_w[0, 0] + att_w[1, 0]) + att_b[0])
    aa = (a * a).reshape(1).astype(jnp.float32)

    zeros64 = jnp.zeros((ROWS_PER_TILE, 64), jnp.float32)
    zeros16 = jnp.zeros((ROWS_PER_TILE, 16), jnp.float32)
    zeros8 = jnp.zeros((ROWS_PER_TILE, DEG_W), jnp.float32)
    ones8 = jnp.ones((S, DEG_W), jnp.float32)

    dp = _deg_kernel()(dstp, ones8, zeros8)
    ys, s8 = _tc1(x, W1, dp)

    p = _prop_kernel(64)(ys, srcp, dstp, zeros64)
    z1, ys2 = _tc_combine(p, s8, 64)
    q = _prop_kernel(64)(ys2, srcp, dstp, zeros64)
    us = _tc_mid(q, z1, s8, b1.reshape(1, 64), W2, aa)

    r = _prop_kernel(16)(us, srcp, dstp, zeros16)
    v1, us2 = _tc_combine(r, s8, 16)
    t = _prop_kernel(16)(us2, srcp, dstp, zeros16)
    return _tc_final(t, v1, s8, b2.reshape(1, 16), aa)


# speedup vs baseline: 1.1070x; 1.0186x over previous
import functools

import jax
import jax.numpy as jnp
from jax import lax
from jax.experimental import pallas as pl
from jax.experimental.pallas import tpu as pltpu
from jax.experimental.pallas import tpu_sc as plsc

N = 10000
E = 320000
NW = 32
NT = 16
CH = 128
CPW = 80
E_PAD = NW * CPW * CH
N_ACC = 10240
ROWS_PER_TILE = N_ACC // NT
NCOPY = ROWS_PER_TILE // CH
DUMP = 10016
DEG_W = 8
EPW = CPW * CH
S = 1024


def _mesh():
    return plsc.VectorSubcoreMesh(core_axis_name="c", subcore_axis_name="s")


@functools.cache
def _prop_kernel(d):
    @functools.partial(
        pl.kernel,
        out_type=jax.ShapeDtypeStruct((2, N_ACC, d), jnp.float32),
        mesh=_mesh(),
        scratch_types=[
            pltpu.VMEM((EPW,), jnp.int32),
            pltpu.VMEM((EPW,), jnp.int32),
            pltpu.VMEM((S, d), jnp.float32),
            pltpu.VMEM_SHARED((N_ACC, d), jnp.float32),
            pltpu.SemaphoreType.DMA,
        ],
        compiler_params=pltpu.CompilerParams(use_tc_tiling_on_sc=False),
    )
    def prop(hs, srcp, dstp, ztile, out, src_v, dst_v, rows, acc, sem):
        c = lax.axis_index("c")
        t = lax.axis_index("s")
        wid = c * NT + t
        pltpu.sync_copy(ztile, acc.at[pl.ds(t * ROWS_PER_TILE, ROWS_PER_TILE)])
        plsc.subcore_barrier()
        pltpu.sync_copy(srcp.at[wid], src_v)
        pltpu.sync_copy(dstp.at[wid], dst_v)

        def body(g, carry):
            j = S * g
            pltpu.async_copy(hs.at[src_v.at[pl.ds(j, S)]], rows, sem).wait()
            pltpu.sync_copy(rows, acc.at[dst_v.at[pl.ds(j, S)]], add=True)
            return carry

        lax.fori_loop(0, EPW // S, body, 0)
        plsc.subcore_barrier()
        off = t * ROWS_PER_TILE
        pltpu.sync_copy(acc.at[pl.ds(off, ROWS_PER_TILE)],
                        out.at[c, pl.ds(off, ROWS_PER_TILE)])

    return prop


@functools.cache
def _deg_kernel():
    @functools.partial(
        pl.kernel,
        out_type=jax.ShapeDtypeStruct((2, N_ACC, DEG_W), jnp.float32),
        mesh=_mesh(),
        scratch_types=[
            pltpu.VMEM((EPW,), jnp.int32),
            pltpu.VMEM((S, DEG_W), jnp.float32),
            pltpu.VMEM_SHARED((N_ACC, DEG_W), jnp.float32),
        ],
        compiler_params=pltpu.CompilerParams(use_tc_tiling_on_sc=False),
    )
    def degk(dstp, ones_hbm, ztile, out, dst_v, ones_v, acc):
        c = lax.axis_index("c")
        t = lax.axis_index("s")
        wid = c * NT + t
        pltpu.sync_copy(ztile, acc.at[pl.ds(t * ROWS_PER_TILE, ROWS_PER_TILE)])
        plsc.subcore_barrier()
        pltpu.sync_copy(dstp.at[wid], dst_v)
        pltpu.sync_copy(ones_hbm, ones_v)

        def body(g, carry):
            pltpu.sync_copy(ones_v, acc.at[dst_v.at[pl.ds(S * g, S)]], add=True)
            return carry

        lax.fori_loop(0, EPW // S, body, 0)
        plsc.subcore_barrier()
        off = t * ROWS_PER_TILE
        pltpu.sync_copy(acc.at[pl.ds(off, ROWS_PER_TILE)],
                        out.at[c, pl.ds(off, ROWS_PER_TILE)])

    return degk


def _tc1(x, W1, dp):
    def body(x_ref, w_ref, dp_ref, ys_ref, s8_ref):
        deg = (dp_ref[0] + dp_ref[1])[:N]
        s8 = jnp.where(deg > 0, lax.rsqrt(jnp.maximum(deg, 1e-12)), 0.0)
        s8_ref[...] = s8
        y = jnp.dot(x_ref[...], w_ref[...], preferred_element_type=jnp.float32)
        ys_ref[...] = y * s8[:, :1]

    return pl.pallas_call(
        body,
        out_shape=(
            jax.ShapeDtypeStruct((N, 64), jnp.float32),
            jax.ShapeDtypeStruct((N, DEG_W), jnp.float32),
        ),
    )(x, W1, dp)


def _tc_combine(p, s8, d):
    def body(p_ref, s8_ref, z_ref, zs_ref):
        s = s8_ref[...][:, :1]
        z = s * (p_ref[0] + p_ref[1])[:N]
        z_ref[...] = z
        zs_ref[...] = s * z

    return pl.pallas_call(
        body,
        out_shape=(
            jax.ShapeDtypeStruct((N, d), jnp.float32),
            jax.ShapeDtypeStruct((N, d), jnp.float32),
        ),
    )(p, s8)


def _tc_mid(q, z1, s8, b1, W2, aa):
    def body(q_ref, z1_ref, s8_ref, b1_ref, w2_ref, aa_ref, us_ref):
        s = s8_ref[...][:, :1]
        z2 = s * (q_ref[0] + q_ref[1])[:N]
        h = jnp.maximum(0.5 * z1_ref[...] + (0.25 * aa_ref[0]) * z2 + b1_ref[...], 0.0)
        u = jnp.dot(h, w2_ref[...], preferred_element_type=jnp.float32)
        us_ref[...] = s * u

    return pl.pallas_call(
        body,
        in_specs=[
            pl.BlockSpec(memory_space=pltpu.VMEM),
            pl.BlockSpec(memory_space=pltpu.VMEM),
            pl.BlockSpec(memory_space=pltpu.VMEM),
            pl.BlockSpec(memory_space=pltpu.VMEM),
            pl.BlockSpec(memory_space=pltpu.VMEM),
            pl.BlockSpec(memory_space=pltpu.SMEM),
        ],
        out_shape=jax.ShapeDtypeStruct((N, 16), jnp.float32),
    )(q, z1, s8, b1, W2, aa)


def _tc_final(t, v1, s8, b2, aa):
    def body(t_ref, v1_ref, s8_ref, b2_ref, aa_ref, o_ref):
        s = s8_ref[...][:, :1]
        v2 = s * (t_ref[0] + t_ref[1])[:N]
        o = 0.5 * v1_ref[...] + (0.25 * aa_ref[0]) * v2 + b2_ref[...]
        m = jnp.max(o, axis=1, keepdims=True)
        lse = jnp.log(jnp.sum(jnp.exp(o - m), axis=1, keepdims=True)) + m
        o_ref[...] = o - lse

    return pl.pallas_call(
        body,
        in_specs=[
            pl.BlockSpec(memory_space=pltpu.VMEM),
            pl.BlockSpec(memory_space=pltpu.VMEM),
            pl.BlockSpec(memory_space=pltpu.VMEM),
            pl.BlockSpec(memory_space=pltpu.VMEM),
            pl.BlockSpec(memory_space=pltpu.SMEM),
        ],
        out_shape=jax.ShapeDtypeStruct((N, 16), jnp.float32),
    )(t, v1, s8, b2, aa)


def kernel(x, edge_index, W1, b1, W2, b2, att_w, att_b):
    src = edge_index[0].astype(jnp.int32)
    dst = edge_index[1].astype(jnp.int32)
    pad = E_PAD - E
    srcp = jnp.concatenate([src, jnp.zeros((pad,), jnp.int32)]).reshape(NW, EPW)
    pad_dst = N + (jnp.arange(pad, dtype=jnp.int32) % (N_ACC - N))
    dstp = jnp.concatenate([dst, pad_dst]).reshape(NW, EPW)

    a = jax.nn.sigmoid(jax.nn.relu(att_w[0, 0] + att_w[1, 0]) + att_b[0])
    aa = (a * a).reshape(1).astype(jnp.float32)

    zeros64 = jnp.zeros((ROWS_PER_TILE, 64), jnp.float32)
    zeros16 = jnp.zeros((ROWS_PER_TILE, 16), jnp.float32)
    zeros8 = jnp.zeros((ROWS_PER_TILE, DEG_W), jnp.float32)
    ones8 = jnp.ones((S, DEG_W), jnp.float32)

    dp = _deg_kernel()(dstp, ones8, zeros8)
    ys, s8 = _tc1(x, W1, dp)

    p = _prop_kernel(64)(ys, srcp, dstp, zeros64)
    z1, ys2 = _tc_combine(p, s8, 64)
    q = _prop_kernel(64)(ys2, srcp, dstp, zeros64)
    us = _tc_mid(q, z1, s8, b1.reshape(1, 64), W2, aa)

    r = _prop_kernel(16)(us, srcp, dstp, zeros16)
    v1, us2 = _tc_combine(r, s8, 16)
    t = _prop_kernel(16)(us2, srcp, dstp, zeros16)
    return _tc_final(t, v1, s8, b2.reshape(1, 16), aa)


# speedup vs baseline: 1.2218x; 1.1038x over previous
import functools

import jax
import jax.numpy as jnp
from jax import lax
from jax.experimental import pallas as pl
from jax.experimental.pallas import tpu as pltpu
from jax.experimental.pallas import tpu_sc as plsc

N = 10000
E = 320000
NW = 32
NT = 16
CH = 128
CPW = 80
E_PAD = NW * CPW * CH
N_ACC = 10240
ROWS_PER_TILE = N_ACC // NT
NCOPY = ROWS_PER_TILE // CH
DUMP = 10016
DEG_W = 8
EPW = CPW * CH
S = 1024


def _mesh():
    return plsc.VectorSubcoreMesh(core_axis_name="c", subcore_axis_name="s")


@functools.cache
def _prop_kernel(d):
    stage = d <= 16
    scratch = [
        pltpu.VMEM((EPW,), jnp.int32),
        pltpu.VMEM((EPW,), jnp.int32),
        pltpu.VMEM((S, d), jnp.float32),
        pltpu.VMEM_SHARED((N_ACC, d), jnp.float32),
    ]
    if stage:
        scratch.append(pltpu.VMEM_SHARED((N, d), jnp.float32))
    scratch.append(pltpu.SemaphoreType.DMA)

    @functools.partial(
        pl.kernel,
        out_type=pltpu.HBM((2, N_ACC, d), jnp.float32),
        mesh=_mesh(),
        scratch_types=scratch,
        compiler_params=pltpu.CompilerParams(use_tc_tiling_on_sc=False),
    )
    def prop(hs, srcp, dstp, ztile, out, src_v, dst_v, rows, *rest):
        if stage:
            acc, hsp, sem = rest
        else:
            acc, sem = rest
            hsp = None
        c = lax.axis_index("c")
        t = lax.axis_index("s")
        wid = c * NT + t
        pltpu.sync_copy(ztile, acc.at[pl.ds(t * ROWS_PER_TILE, ROWS_PER_TILE)])
        if stage:
            pltpu.sync_copy(hs.at[pl.ds(t * (N // NT), N // NT)],
                            hsp.at[pl.ds(t * (N // NT), N // NT)])
        plsc.subcore_barrier()
        pltpu.sync_copy(srcp.at[wid], src_v)
        pltpu.sync_copy(dstp.at[wid], dst_v)

        gsrc = hsp if stage else hs

        def body(g, carry):
            j = S * g
            pltpu.async_copy(gsrc.at[src_v.at[pl.ds(j, S)]], rows, sem).wait()
            pltpu.sync_copy(rows, acc.at[dst_v.at[pl.ds(j, S)]], add=True)
            return carry

        lax.fori_loop(0, EPW // S, body, 0)
        plsc.subcore_barrier()
        off = t * ROWS_PER_TILE
        pltpu.sync_copy(acc.at[pl.ds(off, ROWS_PER_TILE)],
                        out.at[c, pl.ds(off, ROWS_PER_TILE)])

    return prop


@functools.cache
def _deg_kernel():
    @functools.partial(
        pl.kernel,
        out_type=jax.ShapeDtypeStruct((2, N_ACC, DEG_W), jnp.float32),
        mesh=_mesh(),
        scratch_types=[
            pltpu.VMEM((EPW,), jnp.int32),
            pltpu.VMEM((S, DEG_W), jnp.float32),
            pltpu.VMEM_SHARED((N_ACC, DEG_W), jnp.float32),
        ],
        compiler_params=pltpu.CompilerParams(use_tc_tiling_on_sc=False),
    )
    def degk(dstp, ones_hbm, ztile, out, dst_v, ones_v, acc):
        c = lax.axis_index("c")
        t = lax.axis_index("s")
        wid = c * NT + t
        pltpu.sync_copy(ztile, acc.at[pl.ds(t * ROWS_PER_TILE, ROWS_PER_TILE)])
        plsc.subcore_barrier()
        pltpu.sync_copy(dstp.at[wid], dst_v)
        pltpu.sync_copy(ones_hbm, ones_v)

        def body(g, carry):
            pltpu.sync_copy(ones_v, acc.at[dst_v.at[pl.ds(S * g, S)]], add=True)
            return carry

        lax.fori_loop(0, EPW // S, body, 0)
        plsc.subcore_barrier()
        off = t * ROWS_PER_TILE
        pltpu.sync_copy(acc.at[pl.ds(off, ROWS_PER_TILE)],
                        out.at[c, pl.ds(off, ROWS_PER_TILE)])

    return degk


def _tc1(x, W1, dp):
    def body(x_ref, w_ref, dp_ref, ys_ref, s8_ref):
        deg = (dp_ref[0] + dp_ref[1])[:N]
        s8 = jnp.where(deg > 0, lax.rsqrt(jnp.maximum(deg, 1e-12)), 0.0)
        s8_ref[...] = s8
        y = jnp.dot(x_ref[...], w_ref[...], preferred_element_type=jnp.float32)
        ys_ref[...] = y * s8[:, :1]

    return pl.pallas_call(
        body,
        out_shape=(
            jax.ShapeDtypeStruct((N, 64), jnp.float32),
            jax.ShapeDtypeStruct((N, DEG_W), jnp.float32),
        ),
    )(x, W1, dp)


def _tc_combine(p, s8, d):
    def body(p_ref, s8_ref, z_ref, zs_ref):
        s = s8_ref[...][:, :1]
        z = s * (p_ref[0] + p_ref[1])[:N]
        z_ref[...] = z
        zs_ref[...] = s * z

    return pl.pallas_call(
        body,
        out_shape=(
            jax.ShapeDtypeStruct((N, d), jnp.float32),
            jax.ShapeDtypeStruct((N, d), jnp.float32),
        ),
    )(p, s8)


def _tc_mid(q, z1, s8, b1, W2, aa):
    def body(q_ref, z1_ref, s8_ref, b1_ref, w2_ref, aa_ref, us_ref):
        s = s8_ref[...][:, :1]
        z2 = s * (q_ref[0] + q_ref[1])[:N]
        h = jnp.maximum(0.5 * z1_ref[...] + (0.25 * aa_ref[0]) * z2 + b1_ref[...], 0.0)
        u = jnp.dot(h, w2_ref[...], preferred_element_type=jnp.float32)
        us_ref[...] = s * u

    return pl.pallas_call(
        body,
        in_specs=[
            pl.BlockSpec(memory_space=pltpu.VMEM),
            pl.BlockSpec(memory_space=pltpu.VMEM),
            pl.BlockSpec(memory_space=pltpu.VMEM),
            pl.BlockSpec(memory_space=pltpu.VMEM),
            pl.BlockSpec(memory_space=pltpu.VMEM),
            pl.BlockSpec(memory_space=pltpu.SMEM),
        ],
        out_shape=jax.ShapeDtypeStruct((N, 16), jnp.float32),
    )(q, z1, s8, b1, W2, aa)


def _tc_final(t, v1, s8, b2, aa):
    def body(t_ref, v1_ref, s8_ref, b2_ref, aa_ref, o_ref):
        s = s8_ref[...][:, :1]
        v2 = s * (t_ref[0] + t_ref[1])[:N]
        o = 0.5 * v1_ref[...] + (0.25 * aa_ref[0]) * v2 + b2_ref[...]
        m = jnp.max(o, axis=1, keepdims=True)
        lse = jnp.log(jnp.sum(jnp.exp(o - m), axis=1, keepdims=True)) + m
        o_ref[...] = o - lse

    return pl.pallas_call(
        body,
        in_specs=[
            pl.BlockSpec(memory_space=pltpu.VMEM),
            pl.BlockSpec(memory_space=pltpu.VMEM),
            pl.BlockSpec(memory_space=pltpu.VMEM),
            pl.BlockSpec(memory_space=pltpu.VMEM),
            pl.BlockSpec(memory_space=pltpu.SMEM),
        ],
        out_shape=jax.ShapeDtypeStruct((N, 16), jnp.float32),
    )(t, v1, s8, b2, aa)


def kernel(x, edge_index, W1, b1, W2, b2, att_w, att_b):
    src = edge_index[0].astype(jnp.int32)
    dst = edge_index[1].astype(jnp.int32)
    pad = E_PAD - E
    srcp = jnp.concatenate([src, jnp.zeros((pad,), jnp.int32)]).reshape(NW, EPW)
    pad_dst = N + (jnp.arange(pad, dtype=jnp.int32) % (N_ACC - N))
    dstp = jnp.concatenate([dst, pad_dst]).reshape(NW, EPW)

    a = jax.nn.sigmoid(jax.nn.relu(att_w[0, 0] + att_w[1, 0]) + att_b[0])
    aa = (a * a).reshape(1).astype(jnp.float32)

    zeros64 = jnp.zeros((ROWS_PER_TILE, 64), jnp.float32)
    zeros16 = jnp.zeros((ROWS_PER_TILE, 16), jnp.float32)
    zeros8 = jnp.zeros((ROWS_PER_TILE, DEG_W), jnp.float32)
    ones8 = jnp.ones((S, DEG_W), jnp.float32)

    dp = _deg_kernel()(dstp, ones8, zeros8)
    ys, s8 = _tc1(x, W1, dp)

    p = _prop_kernel(64)(ys, srcp, dstp, zeros64)
    z1, ys2 = _tc_combine(p, s8, 64)
    q = _prop_kernel(64)(ys2, srcp, dstp, zeros64)
    us = _tc_mid(q, z1, s8, b1.reshape(1, 64), W2, aa)

    r = _prop_kernel(16)(us, srcp, dstp, zeros16)
    v1, us2 = _tc_combine(r, s8, 16)
    t = _prop_kernel(16)(us2, srcp, dstp, zeros16)
    return _tc_final(t, v1, s8, b2.reshape(1, 16), aa)


# speedup vs baseline: 1.9837x; 1.6235x over previous
import functools

import jax
import jax.numpy as jnp
from jax import lax
from jax.experimental import pallas as pl
from jax.experimental.pallas import tpu as pltpu
from jax.experimental.pallas import tpu_sc as plsc

N = 10000
E = 320000
NW = 32
NT = 16
CH = 128
CPW = 80
E_PAD = NW * CPW * CH
N_ACC = 10240
ROWS_PER_TILE = N_ACC // NT
NCOPY = ROWS_PER_TILE // CH
DUMP = 10016
DEG_W = 8
EPW = CPW * CH
S = 1024


def _mesh():
    return plsc.VectorSubcoreMesh(core_axis_name="c", subcore_axis_name="s")


@functools.cache
def _prop_kernel(d):
    stage = d <= 16
    scratch = [
        pltpu.VMEM((EPW,), jnp.int32),
        pltpu.VMEM((EPW,), jnp.int32),
        pltpu.VMEM((S, d), jnp.float32),
        pltpu.VMEM_SHARED((N_ACC, d), jnp.float32),
    ]
    if stage:
        scratch.append(pltpu.VMEM_SHARED((N, d), jnp.float32))
    scratch.append(pltpu.SemaphoreType.DMA)

    @functools.partial(
        pl.kernel,
        out_type=pltpu.HBM((2, N_ACC, d), jnp.float32),
        mesh=_mesh(),
        scratch_types=scratch,
        compiler_params=pltpu.CompilerParams(use_tc_tiling_on_sc=False),
    )
    def prop(hs, srcp, dstp, ztile, out, src_v, dst_v, rows, *rest):
        if stage:
            acc, hsp, sem = rest
        else:
            acc, sem = rest
            hsp = None
        c = lax.axis_index("c")
        t = lax.axis_index("s")
        wid = c * NT + t
        pltpu.sync_copy(ztile, acc.at[pl.ds(t * ROWS_PER_TILE, ROWS_PER_TILE)])
        if stage:
            pltpu.sync_copy(hs.at[pl.ds(t * (N // NT), N // NT)],
                            hsp.at[pl.ds(t * (N // NT), N // NT)])
        plsc.subcore_barrier()
        pltpu.sync_copy(srcp.at[wid], src_v)
        pltpu.sync_copy(dstp.at[wid], dst_v)

        gsrc = hsp if stage else hs

        def body(g, carry):
            j = S * g
            pltpu.async_copy(gsrc.at[src_v.at[pl.ds(j, S)]], rows, sem).wait()
            pltpu.sync_copy(rows, acc.at[dst_v.at[pl.ds(j, S)]], add=True)
            return carry

        lax.fori_loop(0, EPW // S, body, 0)
        plsc.subcore_barrier()
        off = t * ROWS_PER_TILE
        pltpu.sync_copy(acc.at[pl.ds(off, ROWS_PER_TILE)],
                        out.at[c, pl.ds(off, ROWS_PER_TILE)])

    return prop


@functools.cache
def _prop64_kernel():
    DH = 32

    @functools.partial(
        pl.kernel,
        out_type=pltpu.HBM((2, 2, N_ACC, DH), jnp.float32),
        mesh=_mesh(),
        scratch_types=[
            pltpu.VMEM((EPW,), jnp.int32),
            pltpu.VMEM((EPW,), jnp.int32),
            pltpu.VMEM((S, DH), jnp.float32),
            pltpu.VMEM_SHARED((N_ACC, DH), jnp.float32),
            pltpu.VMEM_SHARED((N, DH), jnp.float32),
            pltpu.SemaphoreType.DMA,
        ],
        compiler_params=pltpu.CompilerParams(use_tc_tiling_on_sc=False),
    )
    def prop(hs_lo, hs_hi, srcp, dstp, ztile, out, src_v, dst_v, rows, acc,
             hsp, sem):
        c = lax.axis_index("c")
        t = lax.axis_index("s")
        wid = c * NT + t
        pltpu.sync_copy(srcp.at[wid], src_v)
        pltpu.sync_copy(dstp.at[wid], dst_v)
        for h, hs in enumerate((hs_lo, hs_hi)):
            pltpu.sync_copy(ztile, acc.at[pl.ds(t * ROWS_PER_TILE, ROWS_PER_TILE)])
            pltpu.sync_copy(hs.at[pl.ds(t * (N // NT), N // NT)],
                            hsp.at[pl.ds(t * (N // NT), N // NT)])
            plsc.subcore_barrier()

            def body(g, carry):
                j = S * g
                pltpu.async_copy(hsp.at[src_v.at[pl.ds(j, S)]], rows, sem).wait()
                pltpu.sync_copy(rows, acc.at[dst_v.at[pl.ds(j, S)]], add=True)
                return carry

            lax.fori_loop(0, EPW // S, body, 0)
            plsc.subcore_barrier()
            off = t * ROWS_PER_TILE
            pltpu.sync_copy(acc.at[pl.ds(off, ROWS_PER_TILE)],
                            out.at[h, c, pl.ds(off, ROWS_PER_TILE)])

    return prop


@functools.cache
def _deg_kernel():
    @functools.partial(
        pl.kernel,
        out_type=jax.ShapeDtypeStruct((2, N_ACC, DEG_W), jnp.float32),
        mesh=_mesh(),
        scratch_types=[
            pltpu.VMEM((EPW,), jnp.int32),
            pltpu.VMEM((S, DEG_W), jnp.float32),
            pltpu.VMEM_SHARED((N_ACC, DEG_W), jnp.float32),
        ],
        compiler_params=pltpu.CompilerParams(use_tc_tiling_on_sc=False),
    )
    def degk(dstp, ones_hbm, ztile, out, dst_v, ones_v, acc):
        c = lax.axis_index("c")
        t = lax.axis_index("s")
        wid = c * NT + t
        pltpu.sync_copy(ztile, acc.at[pl.ds(t * ROWS_PER_TILE, ROWS_PER_TILE)])
        plsc.subcore_barrier()
        pltpu.sync_copy(dstp.at[wid], dst_v)
        pltpu.sync_copy(ones_hbm, ones_v)

        def body(g, carry):
            pltpu.sync_copy(ones_v, acc.at[dst_v.at[pl.ds(S * g, S)]], add=True)
            return carry

        lax.fori_loop(0, EPW // S, body, 0)
        plsc.subcore_barrier()
        off = t * ROWS_PER_TILE
        pltpu.sync_copy(acc.at[pl.ds(off, ROWS_PER_TILE)],
                        out.at[c, pl.ds(off, ROWS_PER_TILE)])

    return degk


def _tc1(x, W1, dp):
    def body(x_ref, w_ref, dp_ref, ys_ref, s8_ref):
        deg = (dp_ref[0] + dp_ref[1])[:N]
        s8 = jnp.where(deg > 0, lax.rsqrt(jnp.maximum(deg, 1e-12)), 0.0)
        s8_ref[...] = s8
        y = jnp.dot(x_ref[...], w_ref[...], preferred_element_type=jnp.float32)
        ys_ref[...] = y * s8[:, :1]

    return pl.pallas_call(
        body,
        out_shape=(
            jax.ShapeDtypeStruct((N, 64), jnp.float32),
            jax.ShapeDtypeStruct((N, DEG_W), jnp.float32),
        ),
    )(x, W1, dp)


def _combine_partials(p_ref):
    if p_ref.ndim == 4:
        return jnp.concatenate(
            [(p_ref[0, 0] + p_ref[0, 1])[:N], (p_ref[1, 0] + p_ref[1, 1])[:N]],
            axis=1)
    return (p_ref[0] + p_ref[1])[:N]


def _tc_combine(p, s8, d):
    def body(p_ref, s8_ref, z_ref, zs_ref):
        s = s8_ref[...][:, :1]
        z = s * _combine_partials(p_ref)
        z_ref[...] = z
        zs_ref[...] = s * z

    return pl.pallas_call(
        body,
        out_shape=(
            jax.ShapeDtypeStruct((N, d), jnp.float32),
            jax.ShapeDtypeStruct((N, d), jnp.float32),
        ),
    )(p, s8)


def _tc_mid(q, z1, s8, b1, W2, aa):
    def body(q_ref, z1_ref, s8_ref, b1_ref, w2_ref, aa_ref, us_ref):
        s = s8_ref[...][:, :1]
        z2 = s * _combine_partials(q_ref)
        h = jnp.maximum(0.5 * z1_ref[...] + (0.25 * aa_ref[0]) * z2 + b1_ref[...], 0.0)
        u = jnp.dot(h, w2_ref[...], preferred_element_type=jnp.float32)
        us_ref[...] = s * u

    return pl.pallas_call(
        body,
        in_specs=[
            pl.BlockSpec(memory_space=pltpu.VMEM),
            pl.BlockSpec(memory_space=pltpu.VMEM),
            pl.BlockSpec(memory_space=pltpu.VMEM),
            pl.BlockSpec(memory_space=pltpu.VMEM),
            pl.BlockSpec(memory_space=pltpu.VMEM),
            pl.BlockSpec(memory_space=pltpu.SMEM),
        ],
        out_shape=jax.ShapeDtypeStruct((N, 16), jnp.float32),
    )(q, z1, s8, b1, W2, aa)


def _tc_final(t, v1, s8, b2, aa):
    def body(t_ref, v1_ref, s8_ref, b2_ref, aa_ref, o_ref):
        s = s8_ref[...][:, :1]
        v2 = s * _combine_partials(t_ref)
        o = 0.5 * v1_ref[...] + (0.25 * aa_ref[0]) * v2 + b2_ref[...]
        m = jnp.max(o, axis=1, keepdims=True)
        lse = jnp.log(jnp.sum(jnp.exp(o - m), axis=1, keepdims=True)) + m
        o_ref[...] = o - lse

    return pl.pallas_call(
        body,
        in_specs=[
            pl.BlockSpec(memory_space=pltpu.VMEM),
            pl.BlockSpec(memory_space=pltpu.VMEM),
            pl.BlockSpec(memory_space=pltpu.VMEM),
            pl.BlockSpec(memory_space=pltpu.VMEM),
            pl.BlockSpec(memory_space=pltpu.SMEM),
        ],
        out_shape=jax.ShapeDtypeStruct((N, 16), jnp.float32),
    )(t, v1, s8, b2, aa)


def kernel(x, edge_index, W1, b1, W2, b2, att_w, att_b):
    src = edge_index[0].astype(jnp.int32)
    dst = edge_index[1].astype(jnp.int32)
    pad = E_PAD - E
    srcp = jnp.concatenate([src, jnp.zeros((pad,), jnp.int32)]).reshape(NW, EPW)
    pad_dst = N + (jnp.arange(pad, dtype=jnp.int32) % (N_ACC - N))
    dstp = jnp.concatenate([dst, pad_dst]).reshape(NW, EPW)

    a = jax.nn.sigmoid(jax.nn.relu(att_w[0, 0] + att_w[1, 0]) + att_b[0])
    aa = (a * a).reshape(1).astype(jnp.float32)

    zeros32 = jnp.zeros((ROWS_PER_TILE, 32), jnp.float32)
    zeros16 = jnp.zeros((ROWS_PER_TILE, 16), jnp.float32)
    zeros8 = jnp.zeros((ROWS_PER_TILE, DEG_W), jnp.float32)
    ones8 = jnp.ones((S, DEG_W), jnp.float32)

    dp = _deg_kernel()(dstp, ones8, zeros8)
    ys, s8 = _tc1(x, W1, dp)

    p = _prop64_kernel()(ys[:, :32], ys[:, 32:], srcp, dstp, zeros32)
    z1, ys2 = _tc_combine(p, s8, 64)
    q = _prop64_kernel()(ys2[:, :32], ys2[:, 32:], srcp, dstp, zeros32)
    us = _tc_mid(q, z1, s8, b1.reshape(1, 64), W2, aa)

    r = _prop_kernel(16)(us, srcp, dstp, zeros16)
    v1, us2 = _tc_combine(r, s8, 16)
    t = _prop_kernel(16)(us2, srcp, dstp, zeros16)
    return _tc_final(t, v1, s8, b2.reshape(1, 16), aa)


# speedup vs baseline: 2.0162x; 1.0164x over previous
import functools

import jax
import jax.numpy as jnp
from jax import lax
from jax.experimental import pallas as pl
from jax.experimental.pallas import tpu as pltpu
from jax.experimental.pallas import tpu_sc as plsc

N = 10000
E = 320000
NW = 32
NT = 16
CH = 128
CPW = 80
E_PAD = NW * CPW * CH
N_ACC = 10240
ROWS_PER_TILE = N_ACC // NT
NCOPY = ROWS_PER_TILE // CH
DUMP = 10016
DEG_W = 8
EPW = CPW * CH
S = 1024


def _mesh():
    return plsc.VectorSubcoreMesh(core_axis_name="c", subcore_axis_name="s")


@functools.cache
def _prop_kernel(d):
    stage = d <= 16
    scratch = [
        pltpu.VMEM((EPW,), jnp.int32),
        pltpu.VMEM((EPW,), jnp.int32),
        pltpu.VMEM((S, d), jnp.float32),
        pltpu.VMEM_SHARED((N_ACC, d), jnp.float32),
    ]
    if stage:
        scratch.append(pltpu.VMEM_SHARED((N, d), jnp.float32))
    scratch.append(pltpu.SemaphoreType.DMA)

    @functools.partial(
        pl.kernel,
        out_type=pltpu.HBM((2, N_ACC, d), jnp.float32),
        mesh=_mesh(),
        scratch_types=scratch,
        compiler_params=pltpu.CompilerParams(use_tc_tiling_on_sc=False),
    )
    def prop(hs, srcp, dstp, ztile, out, src_v, dst_v, rows, *rest):
        if stage:
            acc, hsp, sem = rest
        else:
            acc, sem = rest
            hsp = None
        c = lax.axis_index("c")
        t = lax.axis_index("s")
        wid = c * NT + t
        pltpu.sync_copy(ztile, acc.at[pl.ds(t * ROWS_PER_TILE, ROWS_PER_TILE)])
        if stage:
            pltpu.sync_copy(hs.at[pl.ds(t * (N // NT), N // NT)],
                            hsp.at[pl.ds(t * (N // NT), N // NT)])
        plsc.subcore_barrier()
        pltpu.sync_copy(srcp.at[wid], src_v)
        pltpu.sync_copy(dstp.at[wid], dst_v)

        gsrc = hsp if stage else hs

        def body(g, carry):
            j = S * g
            pltpu.async_copy(gsrc.at[src_v.at[pl.ds(j, S)]], rows, sem).wait()
            pltpu.sync_copy(rows, acc.at[dst_v.at[pl.ds(j, S)]], add=True)
            return carry

        lax.fori_loop(0, EPW // S, body, 0)
        plsc.subcore_barrier()
        off = t * ROWS_PER_TILE
        pltpu.sync_copy(acc.at[pl.ds(off, ROWS_PER_TILE)],
                        out.at[c, pl.ds(off, ROWS_PER_TILE)])

    return prop


@functools.cache
def _prop64_kernel():
    DH = 32

    @functools.partial(
        pl.kernel,
        out_type=pltpu.HBM((2, 2, N_ACC, DH), jnp.float32),
        mesh=_mesh(),
        scratch_types=[
            pltpu.VMEM((EPW,), jnp.int32),
            pltpu.VMEM((EPW,), jnp.int32),
            pltpu.VMEM((S, DH), jnp.float32),
            pltpu.VMEM_SHARED((N_ACC, DH), jnp.float32),
            pltpu.VMEM_SHARED((N, DH), jnp.float32),
            pltpu.SemaphoreType.DMA,
        ],
        compiler_params=pltpu.CompilerParams(use_tc_tiling_on_sc=False),
    )
    def prop(hs_lo, hs_hi, srcp, dstp, ztile, out, src_v, dst_v, rows, acc,
             hsp, sem):
        c = lax.axis_index("c")
        t = lax.axis_index("s")
        wid = c * NT + t
        pltpu.sync_copy(srcp.at[wid], src_v)
        pltpu.sync_copy(dstp.at[wid], dst_v)
        for h, hs in enumerate((hs_lo, hs_hi)):
            pltpu.sync_copy(ztile, acc.at[pl.ds(t * ROWS_PER_TILE, ROWS_PER_TILE)])
            pltpu.sync_copy(hs.at[pl.ds(t * (N // NT), N // NT)],
                            hsp.at[pl.ds(t * (N // NT), N // NT)])
            plsc.subcore_barrier()

            def body(g, carry):
                j = S * g
                pltpu.async_copy(hsp.at[src_v.at[pl.ds(j, S)]], rows, sem).wait()
                pltpu.sync_copy(rows, acc.at[dst_v.at[pl.ds(j, S)]], add=True)
                return carry

            lax.fori_loop(0, EPW // S, body, 0)
            plsc.subcore_barrier()
            off = t * ROWS_PER_TILE
            pltpu.sync_copy(acc.at[pl.ds(off, ROWS_PER_TILE)],
                            out.at[h, c, pl.ds(off, ROWS_PER_TILE)])

    return prop


@functools.cache
def _deg_kernel():
    @functools.partial(
        pl.kernel,
        out_type=jax.ShapeDtypeStruct((2, N_ACC, DEG_W), jnp.float32),
        mesh=_mesh(),
        scratch_types=[
            pltpu.VMEM((EPW,), jnp.int32),
            pltpu.VMEM((S, DEG_W), jnp.float32),
            pltpu.VMEM_SHARED((N_ACC, DEG_W), jnp.float32),
        ],
        compiler_params=pltpu.CompilerParams(use_tc_tiling_on_sc=False),
    )
    def degk(dstp, ones_hbm, ztile, out, dst_v, ones_v, acc):
        c = lax.axis_index("c")
        t = lax.axis_index("s")
        wid = c * NT + t
        pltpu.sync_copy(ztile, acc.at[pl.ds(t * ROWS_PER_TILE, ROWS_PER_TILE)])
        plsc.subcore_barrier()
        pltpu.sync_copy(dstp.at[wid], dst_v)
        pltpu.sync_copy(ones_hbm, ones_v)

        def body(g, carry):
            pltpu.sync_copy(ones_v, acc.at[dst_v.at[pl.ds(S * g, S)]], add=True)
            return carry

        lax.fori_loop(0, EPW // S, body, 0)
        plsc.subcore_barrier()
        off = t * ROWS_PER_TILE
        pltpu.sync_copy(acc.at[pl.ds(off, ROWS_PER_TILE)],
                        out.at[c, pl.ds(off, ROWS_PER_TILE)])

    return degk


def _tc1(x, W1, dp):
    def body(x_ref, w_ref, dp_ref, ylo_ref, yhi_ref, s8_ref):
        deg = (dp_ref[0] + dp_ref[1])[:N]
        s8 = jnp.where(deg > 0, lax.rsqrt(jnp.maximum(deg, 1e-12)), 0.0)
        s8_ref[...] = s8
        s = s8[:, :1]
        xv = x_ref[...]
        ylo_ref[...] = s * jnp.dot(xv, w_ref[...][:, :32],
                                   preferred_element_type=jnp.float32)
        yhi_ref[...] = s * jnp.dot(xv, w_ref[...][:, 32:],
                                   preferred_element_type=jnp.float32)

    return pl.pallas_call(
        body,
        out_shape=(
            jax.ShapeDtypeStruct((N, 32), jnp.float32),
            jax.ShapeDtypeStruct((N, 32), jnp.float32),
            jax.ShapeDtypeStruct((N, DEG_W), jnp.float32),
        ),
    )(x, W1, dp)


def _tc_combine64(p, s8):
    def body(p_ref, s8_ref, zlo_ref, zhi_ref, zslo_ref, zshi_ref):
        s = s8_ref[...][:, :1]
        zlo = s * (p_ref[0, 0] + p_ref[0, 1])[:N]
        zhi = s * (p_ref[1, 0] + p_ref[1, 1])[:N]
        zlo_ref[...] = zlo
        zhi_ref[...] = zhi
        zslo_ref[...] = s * zlo
        zshi_ref[...] = s * zhi

    return pl.pallas_call(
        body,
        out_shape=tuple(jax.ShapeDtypeStruct((N, 32), jnp.float32)
                        for _ in range(4)),
    )(p, s8)


def _tc_combine16(r, s8):
    def body(r_ref, s8_ref, z_ref, zs_ref):
        s = s8_ref[...][:, :1]
        z = s * (r_ref[0] + r_ref[1])[:N]
        z_ref[...] = z
        zs_ref[...] = s * z

    return pl.pallas_call(
        body,
        out_shape=(
            jax.ShapeDtypeStruct((N, 16), jnp.float32),
            jax.ShapeDtypeStruct((N, 16), jnp.float32),
        ),
    )(r, s8)


def _tc_mid(q, z1lo, z1hi, s8, b1, W2, aa):
    def body(q_ref, z1lo_ref, z1hi_ref, s8_ref, b1_ref, w2_ref, aa_ref,
             us_ref):
        s = s8_ref[...][:, :1]
        cc = 0.25 * aa_ref[0]
        b1v = b1_ref[...]
        w2v = w2_ref[...]
        hlo = jnp.maximum(0.5 * z1lo_ref[...] + (cc * s) * (q_ref[0, 0] + q_ref[0, 1])[:N]
                          + b1v[:, :32], 0.0)
        hhi = jnp.maximum(0.5 * z1hi_ref[...] + (cc * s) * (q_ref[1, 0] + q_ref[1, 1])[:N]
                          + b1v[:, 32:], 0.0)
        u = (jnp.dot(hlo, w2v[:32], preferred_element_type=jnp.float32)
             + jnp.dot(hhi, w2v[32:], preferred_element_type=jnp.float32))
        us_ref[...] = s * u

    return pl.pallas_call(
        body,
        in_specs=[
            pl.BlockSpec(memory_space=pltpu.VMEM),
            pl.BlockSpec(memory_space=pltpu.VMEM),
            pl.BlockSpec(memory_space=pltpu.VMEM),
            pl.BlockSpec(memory_space=pltpu.VMEM),
            pl.BlockSpec(memory_space=pltpu.VMEM),
            pl.BlockSpec(memory_space=pltpu.VMEM),
            pl.BlockSpec(memory_space=pltpu.SMEM),
        ],
        out_shape=jax.ShapeDtypeStruct((N, 16), jnp.float32),
    )(q, z1lo, z1hi, s8, b1, W2, aa)


def _tc_final(t, v1, s8, b2, aa):
    def body(t_ref, v1_ref, s8_ref, b2_ref, aa_ref, o_ref):
        s = s8_ref[...][:, :1]
        v2 = s * (t_ref[0] + t_ref[1])[:N]
        o = 0.5 * v1_ref[...] + (0.25 * aa_ref[0]) * v2 + b2_ref[...]
        m = jnp.max(o, axis=1, keepdims=True)
        lse = jnp.log(jnp.sum(jnp.exp(o - m), axis=1, keepdims=True)) + m
        o_ref[...] = o - lse

    return pl.pallas_call(
        body,
        in_specs=[
            pl.BlockSpec(memory_space=pltpu.VMEM),
            pl.BlockSpec(memory_space=pltpu.VMEM),
            pl.BlockSpec(memory_space=pltpu.VMEM),
            pl.BlockSpec(memory_space=pltpu.VMEM),
            pl.BlockSpec(memory_space=pltpu.SMEM),
        ],
        out_shape=jax.ShapeDtypeStruct((N, 16), jnp.float32),
    )(t, v1, s8, b2, aa)


def kernel(x, edge_index, W1, b1, W2, b2, att_w, att_b):
    src = edge_index[0].astype(jnp.int32)
    dst = edge_index[1].astype(jnp.int32)
    pad = E_PAD - E
    srcp = jnp.concatenate([src, jnp.zeros((pad,), jnp.int32)]).reshape(NW, EPW)
    pad_dst = N + (jnp.arange(pad, dtype=jnp.int32) % (N_ACC - N))
    dstp = jnp.concatenate([dst, pad_dst]).reshape(NW, EPW)

    a = jax.nn.sigmoid(jax.nn.relu(att_w[0, 0] + att_w[1, 0]) + att_b[0])
    aa = (a * a).reshape(1).astype(jnp.float32)

    zeros32 = jnp.zeros((ROWS_PER_TILE, 32), jnp.float32)
    zeros16 = jnp.zeros((ROWS_PER_TILE, 16), jnp.float32)
    zeros8 = jnp.zeros((ROWS_PER_TILE, DEG_W), jnp.float32)
    ones8 = jnp.ones((S, DEG_W), jnp.float32)

    dp = _deg_kernel()(dstp, ones8, zeros8)
    yslo, yshi, s8 = _tc1(x, W1, dp)

    p = _prop64_kernel()(yslo, yshi, srcp, dstp, zeros32)
    z1lo, z1hi, zslo, zshi = _tc_combine64(p, s8)
    q = _prop64_kernel()(zslo, zshi, srcp, dstp, zeros32)
    us = _tc_mid(q, z1lo, z1hi, s8, b1.reshape(1, 64), W2, aa)

    r = _prop_kernel(16)(us, srcp, dstp, zeros16)
    v1, us2 = _tc_combine16(r, s8)
    t = _prop_kernel(16)(us2, srcp, dstp, zeros16)
    return _tc_final(t, v1, s8, b2.reshape(1, 16), aa)


# speedup vs baseline: 2.0369x; 1.0103x over previous
import functools

import jax
import jax.numpy as jnp
from jax import lax
from jax.experimental import pallas as pl
from jax.experimental.pallas import tpu as pltpu
from jax.experimental.pallas import tpu_sc as plsc

N = 10000
E = 320000
NW = 32
NT = 16
CH = 128
CPW = 80
E_PAD = NW * CPW * CH
N_ACC = 10240
ROWS_PER_TILE = N_ACC // NT
NCOPY = ROWS_PER_TILE // CH
DUMP = 10016
DEG_W = 8
EPW = CPW * CH
S = 1024


def _mesh():
    return plsc.VectorSubcoreMesh(core_axis_name="c", subcore_axis_name="s")


@functools.cache
def _prop_kernel(d):
    stage = d <= 16
    scratch = [
        pltpu.VMEM((EPW,), jnp.int32),
        pltpu.VMEM((EPW,), jnp.int32),
        pltpu.VMEM((S, d), jnp.float32),
        pltpu.VMEM_SHARED((N_ACC, d), jnp.float32),
    ]
    if stage:
        scratch.append(pltpu.VMEM_SHARED((N, d), jnp.float32))
    scratch.append(pltpu.SemaphoreType.DMA)

    @functools.partial(
        pl.kernel,
        out_type=pltpu.HBM((2, N_ACC, d), jnp.float32),
        mesh=_mesh(),
        scratch_types=scratch,
        compiler_params=pltpu.CompilerParams(use_tc_tiling_on_sc=False),
    )
    def prop(hs, srcp, dstp, ztile, out, src_v, dst_v, rows, *rest):
        if stage:
            acc, hsp, sem = rest
        else:
            acc, sem = rest
            hsp = None
        c = lax.axis_index("c")
        t = lax.axis_index("s")
        wid = c * NT + t
        pltpu.sync_copy(ztile, acc.at[pl.ds(t * ROWS_PER_TILE, ROWS_PER_TILE)])
        if stage:
            pltpu.sync_copy(hs.at[pl.ds(t * (N // NT), N // NT)],
                            hsp.at[pl.ds(t * (N // NT), N // NT)])
        plsc.subcore_barrier()
        pltpu.sync_copy(srcp.at[wid], src_v)
        pltpu.sync_copy(dstp.at[wid], dst_v)

        gsrc = hsp if stage else hs

        def body(g, carry):
            j = S * g
            pltpu.async_copy(gsrc.at[src_v.at[pl.ds(j, S)]], rows, sem).wait()
            pltpu.sync_copy(rows, acc.at[dst_v.at[pl.ds(j, S)]], add=True)
            return carry

        lax.fori_loop(0, EPW // S, body, 0)
        plsc.subcore_barrier()
        off = t * ROWS_PER_TILE
        pltpu.sync_copy(acc.at[pl.ds(off, ROWS_PER_TILE)],
                        out.at[c, pl.ds(off, ROWS_PER_TILE)])

    return prop


@functools.cache
def _prop64_kernel():
    DH = 32

    @functools.partial(
        pl.kernel,
        out_type=pltpu.HBM((2, 2, N_ACC, DH), jnp.float32),
        mesh=_mesh(),
        scratch_types=[
            pltpu.VMEM((EPW,), jnp.int32),
            pltpu.VMEM((EPW,), jnp.int32),
            pltpu.VMEM((S, DH), jnp.float32),
            pltpu.VMEM_SHARED((N_ACC, DH), jnp.float32),
            pltpu.VMEM_SHARED((N, DH), jnp.float32),
            pltpu.SemaphoreType.DMA,
        ],
        compiler_params=pltpu.CompilerParams(use_tc_tiling_on_sc=False),
    )
    def prop(hs_lo, hs_hi, srcp, dstp, ztile, out, src_v, dst_v, rows, acc,
             hsp, sem):
        c = lax.axis_index("c")
        t = lax.axis_index("s")
        wid = c * NT + t
        pltpu.sync_copy(srcp.at[wid], src_v)
        pltpu.sync_copy(dstp.at[wid], dst_v)
        for h, hs in enumerate((hs_lo, hs_hi)):
            pltpu.sync_copy(ztile, acc.at[pl.ds(t * ROWS_PER_TILE, ROWS_PER_TILE)])
            pltpu.sync_copy(hs.at[pl.ds(t * (N // NT), N // NT)],
                            hsp.at[pl.ds(t * (N // NT), N // NT)])
            plsc.subcore_barrier()

            def body(g, carry):
                j = S * g
                pltpu.async_copy(hsp.at[src_v.at[pl.ds(j, S)]], rows, sem).wait()
                pltpu.sync_copy(rows, acc.at[dst_v.at[pl.ds(j, S)]], add=True)
                return carry

            lax.fori_loop(0, EPW // S, body, 0)
            plsc.subcore_barrier()
            off = t * ROWS_PER_TILE
            pltpu.sync_copy(acc.at[pl.ds(off, ROWS_PER_TILE)],
                            out.at[h, c, pl.ds(off, ROWS_PER_TILE)])

    return prop


@functools.cache
def _deg_kernel():
    @functools.partial(
        pl.kernel,
        out_type=jax.ShapeDtypeStruct((2, N_ACC, DEG_W), jnp.float32),
        mesh=_mesh(),
        scratch_types=[
            pltpu.VMEM((EPW,), jnp.int32),
            pltpu.VMEM((S, DEG_W), jnp.float32),
            pltpu.VMEM_SHARED((N_ACC, DEG_W), jnp.float32),
        ],
        compiler_params=pltpu.CompilerParams(use_tc_tiling_on_sc=False),
    )
    def degk(dstp, ones_hbm, ztile, out, dst_v, ones_v, acc):
        c = lax.axis_index("c")
        t = lax.axis_index("s")
        wid = c * NT + t
        pltpu.sync_copy(ztile, acc.at[pl.ds(t * ROWS_PER_TILE, ROWS_PER_TILE)])
        plsc.subcore_barrier()
        pltpu.sync_copy(dstp.at[wid], dst_v)
        pltpu.sync_copy(ones_hbm, ones_v)

        def body(g, carry):
            pltpu.sync_copy(ones_v, acc.at[dst_v.at[pl.ds(S * g, S)]], add=True)
            return carry

        lax.fori_loop(0, EPW // S, body, 0)
        plsc.subcore_barrier()
        off = t * ROWS_PER_TILE
        pltpu.sync_copy(acc.at[pl.ds(off, ROWS_PER_TILE)],
                        out.at[c, pl.ds(off, ROWS_PER_TILE)])

    return degk


RB = 2000
_NG = N // RB


def _tc1(x, W1, dp):
    def body(x_ref, w_ref, dp_ref, ylo_ref, yhi_ref, s8_ref):
        deg = dp_ref[0] + dp_ref[1]
        s8 = jnp.where(deg > 0, lax.rsqrt(jnp.maximum(deg, 1e-12)), 0.0)
        s8_ref[...] = s8
        s = s8[:, :1]
        xv = x_ref[...]
        ylo_ref[...] = s * jnp.dot(xv, w_ref[...][:, :32],
                                   preferred_element_type=jnp.float32)
        yhi_ref[...] = s * jnp.dot(xv, w_ref[...][:, 32:],
                                   preferred_element_type=jnp.float32)

    return pl.pallas_call(
        body,
        grid=(_NG,),
        in_specs=[
            pl.BlockSpec((RB, 128), lambda i: (i, 0)),
            pl.BlockSpec((128, 64), lambda i: (0, 0)),
            pl.BlockSpec((2, RB, DEG_W), lambda i: (0, i, 0)),
        ],
        out_specs=(
            pl.BlockSpec((RB, 32), lambda i: (i, 0)),
            pl.BlockSpec((RB, 32), lambda i: (i, 0)),
            pl.BlockSpec((RB, DEG_W), lambda i: (i, 0)),
        ),
        out_shape=(
            jax.ShapeDtypeStruct((N, 32), jnp.float32),
            jax.ShapeDtypeStruct((N, 32), jnp.float32),
            jax.ShapeDtypeStruct((N, DEG_W), jnp.float32),
        ),
    )(x, W1, dp)


def _tc_combine64(p, s8):
    def body(p_ref, s8_ref, zlo_ref, zhi_ref, zslo_ref, zshi_ref):
        s = s8_ref[...][:, :1]
        zlo = s * (p_ref[0, 0] + p_ref[0, 1])
        zhi = s * (p_ref[1, 0] + p_ref[1, 1])
        zlo_ref[...] = zlo
        zhi_ref[...] = zhi
        zslo_ref[...] = s * zlo
        zshi_ref[...] = s * zhi

    return pl.pallas_call(
        body,
        grid=(_NG,),
        in_specs=[
            pl.BlockSpec((2, 2, RB, 32), lambda i: (0, 0, i, 0)),
            pl.BlockSpec((RB, DEG_W), lambda i: (i, 0)),
        ],
        out_specs=tuple(pl.BlockSpec((RB, 32), lambda i: (i, 0))
                        for _ in range(4)),
        out_shape=tuple(jax.ShapeDtypeStruct((N, 32), jnp.float32)
                        for _ in range(4)),
    )(p, s8)


def _tc_combine16(r, s8):
    def body(r_ref, s8_ref, z_ref, zs_ref):
        s = s8_ref[...][:, :1]
        z = s * (r_ref[0] + r_ref[1])
        z_ref[...] = z
        zs_ref[...] = s * z

    return pl.pallas_call(
        body,
        grid=(_NG,),
        in_specs=[
            pl.BlockSpec((2, RB, 16), lambda i: (0, i, 0)),
            pl.BlockSpec((RB, DEG_W), lambda i: (i, 0)),
        ],
        out_specs=(
            pl.BlockSpec((RB, 16), lambda i: (i, 0)),
            pl.BlockSpec((RB, 16), lambda i: (i, 0)),
        ),
        out_shape=(
            jax.ShapeDtypeStruct((N, 16), jnp.float32),
            jax.ShapeDtypeStruct((N, 16), jnp.float32),
        ),
    )(r, s8)


def _tc_mid(q, z1lo, z1hi, s8, b1, W2, aa):
    def body(q_ref, z1lo_ref, z1hi_ref, s8_ref, b1_ref, w2_ref, aa_ref,
             us_ref):
        s = s8_ref[...][:, :1]
        cc = 0.25 * aa_ref[0]
        b1v = b1_ref[...]
        w2v = w2_ref[...]
        hlo = jnp.maximum(0.5 * z1lo_ref[...] + (cc * s) * (q_ref[0, 0] + q_ref[0, 1])
                          + b1v[:, :32], 0.0)
        hhi = jnp.maximum(0.5 * z1hi_ref[...] + (cc * s) * (q_ref[1, 0] + q_ref[1, 1])
                          + b1v[:, 32:], 0.0)
        u = (jnp.dot(hlo, w2v[:32], preferred_element_type=jnp.float32)
             + jnp.dot(hhi, w2v[32:], preferred_element_type=jnp.float32))
        us_ref[...] = s * u

    return pl.pallas_call(
        body,
        grid=(_NG,),
        in_specs=[
            pl.BlockSpec((2, 2, RB, 32), lambda i: (0, 0, i, 0)),
            pl.BlockSpec((RB, 32), lambda i: (i, 0)),
            pl.BlockSpec((RB, 32), lambda i: (i, 0)),
            pl.BlockSpec((RB, DEG_W), lambda i: (i, 0)),
            pl.BlockSpec((1, 64), lambda i: (0, 0)),
            pl.BlockSpec((64, 16), lambda i: (0, 0)),
            pl.BlockSpec(memory_space=pltpu.SMEM),
        ],
        out_specs=pl.BlockSpec((RB, 16), lambda i: (i, 0)),
        out_shape=jax.ShapeDtypeStruct((N, 16), jnp.float32),
    )(q, z1lo, z1hi, s8, b1, W2, aa)


def _tc_final(t, v1, s8, b2, aa):
    def body(t_ref, v1_ref, s8_ref, b2_ref, aa_ref, o_ref):
        s = s8_ref[...][:, :1]
        v2 = s * (t_ref[0] + t_ref[1])
        o = 0.5 * v1_ref[...] + (0.25 * aa_ref[0]) * v2 + b2_ref[...]
        m = jnp.max(o, axis=1, keepdims=True)
        lse = jnp.log(jnp.sum(jnp.exp(o - m), axis=1, keepdims=True)) + m
        o_ref[...] = o - lse

    return pl.pallas_call(
        body,
        grid=(_NG,),
        in_specs=[
            pl.BlockSpec((2, RB, 16), lambda i: (0, i, 0)),
            pl.BlockSpec((RB, 16), lambda i: (i, 0)),
            pl.BlockSpec((RB, DEG_W), lambda i: (i, 0)),
            pl.BlockSpec((1, 16), lambda i: (0, 0)),
            pl.BlockSpec(memory_space=pltpu.SMEM),
        ],
        out_specs=pl.BlockSpec((RB, 16), lambda i: (i, 0)),
        out_shape=jax.ShapeDtypeStruct((N, 16), jnp.float32),
    )(t, v1, s8, b2, aa)


def kernel(x, edge_index, W1, b1, W2, b2, att_w, att_b):
    src = edge_index[0].astype(jnp.int32)
    dst = edge_index[1].astype(jnp.int32)
    pad = E_PAD - E
    srcp = jnp.concatenate([src, jnp.zeros((pad,), jnp.int32)]).reshape(NW, EPW)
    pad_dst = N + (jnp.arange(pad, dtype=jnp.int32) % (N_ACC - N))
    dstp = jnp.concatenate([dst, pad_dst]).reshape(NW, EPW)

    a = jax.nn.sigmoid(jax.nn.relu(att_w[0, 0] + att_w[1, 0]) + att_b[0])
    aa = (a * a).reshape(1).astype(jnp.float32)

    zeros32 = jnp.zeros((ROWS_PER_TILE, 32), jnp.float32)
    zeros16 = jnp.zeros((ROWS_PER_TILE, 16), jnp.float32)
    zeros8 = jnp.zeros((ROWS_PER_TILE, DEG_W), jnp.float32)
    ones8 = jnp.ones((S, DEG_W), jnp.float32)

    dp = _deg_kernel()(dstp, ones8, zeros8)
    yslo, yshi, s8 = _tc1(x, W1, dp)

    p = _prop64_kernel()(yslo, yshi, srcp, dstp, zeros32)
    z1lo, z1hi, zslo, zshi = _tc_combine64(p, s8)
    q = _prop64_kernel()(zslo, zshi, srcp, dstp, zeros32)
    us = _tc_mid(q, z1lo, z1hi, s8, b1.reshape(1, 64), W2, aa)

    r = _prop_kernel(16)(us, srcp, dstp, zeros16)
    v1, us2 = _tc_combine16(r, s8)
    t = _prop_kernel(16)(us2, srcp, dstp, zeros16)
    return _tc_final(t, v1, s8, b2.reshape(1, 16), aa)
